# baseline (speedup 1.0000x reference)
import functools

import jax
import jax.numpy as jnp
from jax import lax
from jax.experimental import pallas as pl
from jax.experimental.pallas import tpu as pltpu
from jax.experimental.pallas import tpu_sc as plsc


@functools.partial(jax.jit, static_argnames=())
def _lookup(indices, embeddings):
    (B,) = indices.shape
    V, D = embeddings.shape
    info = plsc.get_sparse_core_info()
    NW = info.num_cores * info.num_subcores
    assert B % NW == 0
    b_per_w = B // NW
    mesh = plsc.VectorSubcoreMesh(core_axis_name="c", subcore_axis_name="s")

    @functools.partial(
        pl.kernel,
        mesh=mesh,
        out_type=jax.ShapeDtypeStruct((B, D), jnp.float32),
        scratch_types=[
            pltpu.VMEM((b_per_w,), jnp.int32),
            pltpu.VMEM((b_per_w, D), jnp.float32),
            pltpu.SemaphoreType.DMA,
        ],
        compiler_params=pltpu.CompilerParams(use_tc_tiling_on_sc=False),
    )
    def k(table_hbm, idx_hbm, out_hbm, idx_v, rows_v, sem):
        wid = lax.axis_index("s") * info.num_cores + lax.axis_index("c")
        base = wid * b_per_w
        pltpu.sync_copy(idx_hbm.at[pl.ds(base, b_per_w)], idx_v)
        pltpu.async_copy(table_hbm.at[idx_v], rows_v, sem).wait()
        pltpu.sync_copy(rows_v, out_hbm.at[pl.ds(base, b_per_w)])

    return k(embeddings, indices)


def kernel(indices, embeddings):
    return _lookup(indices.astype(jnp.int32), embeddings)


# speedup vs baseline: 3.5321x; 3.5321x over previous
import functools

import jax
import jax.numpy as jnp
from jax import lax
from jax.experimental import pallas as pl
from jax.experimental.pallas import tpu as pltpu
from jax.experimental.pallas import tpu_sc as plsc


def _lookup(idx, tt):
    (B,) = idx.shape
    D, V = tt.shape
    info = plsc.get_sparse_core_info()
    NW = info.num_cores * info.num_subcores
    L = info.num_lanes
    bpw = B // NW
    WAVE = 16
    nwave = bpw // WAVE
    mesh = plsc.VectorSubcoreMesh(core_axis_name="c", subcore_axis_name="s")

    @functools.partial(
        pl.kernel,
        mesh=mesh,
        out_type=jax.ShapeDtypeStruct((D, B), jnp.float32),
        scratch_types=[
            pltpu.VMEM((bpw,), jnp.int32),
            pltpu.VMEM((WAVE, D, 128), jnp.float32),
            pltpu.VMEM((D, bpw), jnp.float32),
            pltpu.SemaphoreType.DMA,
        ],
        compiler_params=pltpu.CompilerParams(needs_layout_passes=False),
    )
    def k(tt_hbm, idx_hbm, out_hbm, idx_v, slab_v, col_v, sem):
        wid = lax.axis_index("s") * info.num_cores + lax.axis_index("c")
        base = wid * bpw
        pltpu.sync_copy(idx_hbm.at[pl.ds(base, bpw)], idx_v)

        def wave(g, carry):
            v = idx_v[pl.ds(g * WAVE, L)]
            w = (v >> 7) << 7
            l = v & 127
            for j in range(WAVE):
                row = pl.multiple_of(w[j], 128)
                pltpu.async_copy(
                    tt_hbm.at[:, pl.ds(row, 128)], slab_v.at[j], sem
                )
            for j in range(WAVE):
                pltpu.make_async_copy(
                    tt_hbm.at[:, pl.ds(0, 128)], slab_v.at[0], sem
                ).wait()
            jbase = g * WAVE
            for j in range(WAVE):
                lane = jnp.full((L,), l[j], jnp.int32)
                sj = jnp.full((L,), j, jnp.int32)
                jcol = jnp.full((L,), jbase + j, jnp.int32)
                for h in range(D // L):
                    dvec = lax.iota(jnp.int32, L) + h * L
                    vals = plsc.load_gather(slab_v, [sj, dvec, lane])
                    plsc.store_scatter(col_v, [dvec, jcol], vals)
            return carry

        lax.fori_loop(0, nwave, wave, 0)
        pltpu.sync_copy(col_v, out_hbm.at[:, pl.ds(base, bpw)])

    return k(tt, idx)


def kernel(indices, embeddings):
    out_t = _lookup(indices.astype(jnp.int32), embeddings.T)
    return out_t.T
